# baseline (speedup 1.0000x reference)
import functools

import jax
import jax.numpy as jnp
from jax import lax
from jax.experimental import pallas as pl
from jax.experimental.pallas import tpu as pltpu
from jax.experimental.pallas import tpu_sc as plsc

N = 10000
E = 320000
D = 128
B = 64
ED = 3 * D

NC = 2
NS = 16
NW = NC * NS
EW = E // NW
K = 80
NCHUNK = EW // K
DH = D // NC
ET = E // NS
NCH2 = ET // K
NB = 5
NRND = NCH2 // NB
NP = 10240
RPT = NP // NS
ZR = 128

_f32 = jnp.float32


def _sc_mesh():
    return plsc.VectorSubcoreMesh(
        core_axis_name="c", subcore_axis_name="s", num_cores=NC, num_subcores=NS
    )


def _zero_vmem_2d(ref, nrows, ncols):
    z = jnp.zeros((16,), _f32)

    @pl.loop(0, nrows)
    def _(i):
        for j in range(ncols // 16):
            ref[i, pl.ds(j * 16, 16)] = z


@functools.cache
def _sc_degree_kernel():
    return functools.partial(
        pl.kernel,
        out_type=jax.ShapeDtypeStruct((NW, NP), _f32),
        mesh=_sc_mesh(),
        scratch_types=[
            pltpu.VMEM((NCHUNK, K), jnp.int32),
            pltpu.VMEM((NP,), _f32),
        ],
        compiler_params=pltpu.CompilerParams(needs_layout_passes=False),
    )(_sc_degree_body)


def _sc_degree(ei4d):
    return _sc_degree_kernel()(ei4d)


def _sc_degree_body(ei_hbm, out_hbm, didx, hist):
    c = lax.axis_index("c")
    s = lax.axis_index("s")
    wid = c * NS + s

    pltpu.sync_copy(ei_hbm.at[1, wid], didx)

    z = jnp.zeros((16,), _f32)

    @pl.loop(0, NP // 16)
    def _(i):
        hist[pl.ds(i * 16, 16)] = z

    one = jnp.ones((16,), _f32)

    @pl.loop(0, NCHUNK)
    def _(i):
        for j in range(K // 16):
            d16 = didx[i, pl.ds(j * 16, 16)]
            plsc.addupdate_scatter(hist, [d16], one)

    pltpu.sync_copy(hist, out_hbm.at[wid])


@functools.cache
def _sc_scatter_kernel():
    return functools.partial(
        pl.kernel,
        out_type=jax.ShapeDtypeStruct((NC, NP, DH), _f32),
        mesh=_sc_mesh(),
        scratch_types=[
            pltpu.VMEM((NCH2, K), jnp.int32),
            pltpu.VMEM((NCH2, K), jnp.int32),
            pltpu.VMEM((NB, K, DH), _f32),
            pltpu.VMEM((ZR, DH), _f32),
            pltpu.VMEM_SHARED((NP, DH), _f32),
        ] + [pltpu.SemaphoreType.DMA] * NB,
        compiler_params=pltpu.CompilerParams(use_tc_tiling_on_sc=False),
    )(_sc_scatter_body)


def _sc_scatter(ei4, m2):
    return _sc_scatter_kernel()(ei4, m2)


def _sc_scatter_body(ei_hbm, m_hbm, out_hbm, sidx, didx, rows,
                     zbuf, acc, *sems):
    c = lax.axis_index("c")
    s = lax.axis_index("s")

    pltpu.sync_copy(ei_hbm.at[0, s], sidx)
    pltpu.sync_copy(ei_hbm.at[1, s], didx)

    _zero_vmem_2d(zbuf, ZR, DH)

    @pl.loop(0, RPT // ZR)
    def _(t):
        pltpu.sync_copy(zbuf, acc.at[pl.ds(s * RPT + t * ZR, ZR)])

    plsc.subcore_barrier()

    for i in range(NB):
        pltpu.async_copy(m_hbm.at[c].at[sidx.at[i]], rows.at[i], sems[i])

    @pl.loop(0, NRND)
    def _(t):
        for i in range(NB):
            j = t * NB + i
            pltpu.make_async_copy(m_hbm.at[c].at[sidx.at[0]],
                                  rows.at[i], sems[i]).wait()
            pltpu.async_copy(rows.at[i], acc.at[didx.at[j]], sems[i],
                             add=True)
            pltpu.make_async_copy(m_hbm.at[c].at[sidx.at[0]],
                                  rows.at[i], sems[i]).wait()
            jn = j + NB

            @pl.when(jn < NCH2)
            def _():
                pltpu.async_copy(m_hbm.at[c].at[sidx.at[jn]],
                                 rows.at[i], sems[i])

    plsc.subcore_barrier()
    pltpu.sync_copy(acc.at[pl.ds(s * RPT, RPT)], out_hbm.at[c, pl.ds(s * RPT, RPT)])


def _dot(a, b):
    return jnp.dot(a, b)


def _expproj(t):
    n = jnp.maximum(jnp.sqrt(jnp.sum(t * t, axis=1, keepdims=True)), 1e-15)
    y = jnp.tanh(n) * t / n
    n2 = jnp.maximum(jnp.sqrt(jnp.sum(y * y, axis=1, keepdims=True)), 1e-15)
    maxn = 1.0 - 4e-3
    return jnp.where(n2 > maxn, y * (maxn / n2), y)


R = 1000
G = N // R
RD = 1024
GD = NP // RD


def _tc_pre_body(x_ref, hist_ref, w1_ref, m1_ref, dinv_ref, dfull_ref):
    i = pl.program_id(0)
    ones = jnp.ones((NW, 1), _f32)
    deg = lax.dot_general(hist_ref[...], ones, (((0,), (0,)), ((), ())),
                          precision=lax.Precision.HIGHEST,
                          preferred_element_type=_f32)
    dfull_ref[...] = lax.rsqrt(deg + 1.0)
    dinv = dfull_ref[pl.ds(i * R, R), :]
    dinv_ref[...] = dinv
    x = x_ref[...]
    n = jnp.maximum(jnp.sqrt(jnp.sum(x * x, axis=1, keepdims=True)), 1e-15)
    z = jnp.clip(n, -1.0 + 1e-7, 1.0 - 1e-7)
    artanh = 0.5 * jnp.log((1.0 + z) / (1.0 - z))
    h0 = artanh * x / n
    mh = dinv * _dot(h0, w1_ref[...])
    m1_ref[0] = mh[:, :DH]
    m1_ref[1] = mh[:, DH:]


def _tc_pre(x, hist, W1):
    return pl.pallas_call(
        _tc_pre_body,
        grid=(G,),
        in_specs=[
            pl.BlockSpec((R, D), lambda i: (i, 0)),
            pl.BlockSpec((NW, NP), lambda i: (0, 0)),
            pl.BlockSpec((D, D), lambda i: (0, 0)),
        ],
        out_specs=[
            pl.BlockSpec((NC, R, DH), lambda i: (0, i, 0)),
            pl.BlockSpec((R, 1), lambda i: (i, 0)),
        ],
        out_shape=[
            jax.ShapeDtypeStruct((NC, N, DH), _f32),
            jax.ShapeDtypeStruct((N, 1), _f32),
        ],
        scratch_shapes=[pltpu.VMEM((NP, 1), _f32)],
    )(x, hist, W1)


def _cat2(r):
    return jnp.concatenate([r[0], r[1]], axis=1)


def _tc_layer_body(acc_ref, m_ref, dinv_ref, w_ref, h_ref, mn_ref):
    dinv = dinv_ref[...]
    h = jnp.maximum(dinv * (_cat2(acc_ref) + _cat2(m_ref)), 0.0)
    h_ref[...] = h
    mh = dinv * _dot(h, w_ref[...])
    mn_ref[0] = mh[:, :DH]
    mn_ref[1] = mh[:, DH:]


def _tc_layer(acc, m, dinv, Wn):
    return pl.pallas_call(
        _tc_layer_body,
        grid=(G,),
        in_specs=[
            pl.BlockSpec((NC, R, DH), lambda i: (0, i, 0)),
            pl.BlockSpec((NC, R, DH), lambda i: (0, i, 0)),
            pl.BlockSpec((R, 1), lambda i: (i, 0)),
            pl.BlockSpec((D, D), lambda i: (0, 0)),
        ],
        out_specs=[
            pl.BlockSpec((R, D), lambda i: (i, 0)),
            pl.BlockSpec((NC, R, DH), lambda i: (0, i, 0)),
        ],
        out_shape=[
            jax.ShapeDtypeStruct((N, D), _f32),
            jax.ShapeDtypeStruct((NC, N, DH), _f32),
        ],
    )(acc, m, dinv, Wn)


def _tc_final_body(acc_ref, m_ref, dinv_ref, h1_ref, h2_ref, b_ref, p1_ref,
                   p2_ref, hypn_ref, gtan_ref):
    i = pl.program_id(0)
    dinv = dinv_ref[...]
    h3 = jnp.maximum(dinv * (_cat2(acc_ref) + _cat2(m_ref)), 0.0)
    ncat = jnp.concatenate([h1_ref[...], h2_ref[...], h3], axis=1)
    t = _dot(jnp.maximum(_dot(ncat, p1_ref[...]), 0.0), p2_ref[...])
    hypn_ref[...] = _expproj(t)

    b = b_ref[0]
    rows = lax.broadcasted_iota(jnp.int32, (B, R), 0)
    oh = (rows == b).astype(_f32)
    gpart = _dot(oh, ncat)

    @pl.when(i == 0)
    def _():
        gtan_ref[...] = gpart

    @pl.when(i > 0)
    def _():
        gtan_ref[...] += gpart


def _tc_final(acc, m3, dinv, h1, h2, batch3, P1, P2):
    return pl.pallas_call(
        _tc_final_body,
        grid=(G,),
        in_specs=[
            pl.BlockSpec((NC, R, DH), lambda i: (0, i, 0)),
            pl.BlockSpec((NC, R, DH), lambda i: (0, i, 0)),
            pl.BlockSpec((R, 1), lambda i: (i, 0)),
            pl.BlockSpec((R, D), lambda i: (i, 0)),
            pl.BlockSpec((R, D), lambda i: (i, 0)),
            pl.BlockSpec((1, 1, R), lambda i: (i, 0, 0)),
            pl.BlockSpec((ED, ED), lambda i: (0, 0)),
            pl.BlockSpec((ED, ED), lambda i: (0, 0)),
        ],
        out_specs=[
            pl.BlockSpec((R, ED), lambda i: (i, 0)),
            pl.BlockSpec((B, ED), lambda i: (0, 0)),
        ],
        out_shape=[
            jax.ShapeDtypeStruct((N, ED), _f32),
            jax.ShapeDtypeStruct((B, ED), _f32),
        ],
    )(acc, m3, dinv, h1, h2, batch3, P1, P2)


def _tc_head_body(g_ref, p1_ref, p2_ref, out_ref):
    t = _dot(jnp.maximum(_dot(g_ref[...], p1_ref[...]), 0.0), p2_ref[...])
    out_ref[...] = _expproj(t)


def _tc_head(gtan, P1, P2):
    return pl.pallas_call(
        _tc_head_body,
        out_shape=jax.ShapeDtypeStruct((B, ED), _f32),
    )(gtan, P1, P2)


def kernel(x, edge_index, batch, W1, W2, W3, P1, P2):
    ei4 = edge_index.reshape(2, NS, NCH2, K)
    ei4d = edge_index.reshape(2, NW, NCHUNK, K)
    batch3 = batch.reshape(G, 1, R)

    hist = _sc_degree(ei4d)
    m1, dinv = _tc_pre(x, hist, W1)
    acc1 = _sc_scatter(ei4, m1)
    h1, m2 = _tc_layer(acc1, m1, dinv, W2)
    acc2 = _sc_scatter(ei4, m2)
    h2, m3 = _tc_layer(acc2, m2, dinv, W3)
    acc3 = _sc_scatter(ei4, m3)
    hyp_n, gtan = _tc_final(acc3, m3, dinv, h1, h2, batch3, P1, P2)
    hyp_g = _tc_head(gtan, P1, P2)
    return (hyp_g, hyp_n)


# speedup vs baseline: 1.0673x; 1.0673x over previous
import functools

import jax
import jax.numpy as jnp
from jax import lax
from jax.experimental import pallas as pl
from jax.experimental.pallas import tpu as pltpu
from jax.experimental.pallas import tpu_sc as plsc

N = 10000
E = 320000
D = 128
B = 64
ED = 3 * D

NC = 2
NS = 16
NW = NC * NS
EW = E // NW
K = 80
NCHUNK = EW // K
DH = D // NC
ET = E // NS
NCH2 = ET // K
NB = 5
NRND = NCH2 // NB
NP = 10240
RPT = NP // NS
ZR = 128

_f32 = jnp.float32


def _sc_mesh():
    return plsc.VectorSubcoreMesh(
        core_axis_name="c", subcore_axis_name="s", num_cores=NC, num_subcores=NS
    )


def _zero_vmem_2d(ref, nrows, ncols):
    z = jnp.zeros((16,), _f32)

    @pl.loop(0, nrows)
    def _(i):
        for j in range(ncols // 16):
            ref[i, pl.ds(j * 16, 16)] = z


@functools.cache
def _sc_degree_kernel():
    return functools.partial(
        pl.kernel,
        out_type=jax.ShapeDtypeStruct((NW, NP), _f32),
        mesh=_sc_mesh(),
        scratch_types=[
            pltpu.VMEM((NCHUNK, K), jnp.int32),
            pltpu.VMEM((NP,), _f32),
        ],
        compiler_params=pltpu.CompilerParams(needs_layout_passes=False),
    )(_sc_degree_body)


def _sc_degree(ei4d):
    return _sc_degree_kernel()(ei4d)


def _sc_degree_body(ei_hbm, out_hbm, didx, hist):
    c = lax.axis_index("c")
    s = lax.axis_index("s")
    wid = c * NS + s

    pltpu.sync_copy(ei_hbm.at[1, wid], didx)

    z = jnp.zeros((16,), _f32)

    @pl.loop(0, NP // 16)
    def _(i):
        hist[pl.ds(i * 16, 16)] = z

    one = jnp.ones((16,), _f32)

    @pl.loop(0, NCHUNK)
    def _(i):
        for j in range(K // 16):
            d16 = didx[i, pl.ds(j * 16, 16)]
            plsc.addupdate_scatter(hist, [d16], one)

    pltpu.sync_copy(hist, out_hbm.at[wid])


@functools.cache
def _sc_scatter_kernel():
    return functools.partial(
        pl.kernel,
        out_type=jax.ShapeDtypeStruct((NC, NP, DH), _f32),
        mesh=_sc_mesh(),
        scratch_types=[
            pltpu.VMEM((NCH2, K), jnp.int32),
            pltpu.VMEM((NCH2, K), jnp.int32),
            pltpu.VMEM((NB, K, DH), _f32),
            pltpu.VMEM((ZR, DH), _f32),
            pltpu.VMEM_SHARED((NP, DH), _f32),
        ] + [pltpu.SemaphoreType.DMA] * NB,
        compiler_params=pltpu.CompilerParams(use_tc_tiling_on_sc=False),
    )(_sc_scatter_body)


def _sc_scatter(ei4, m2):
    return _sc_scatter_kernel()(ei4, m2)


def _sc_scatter_body(ei_hbm, m_hbm, out_hbm, sidx, didx, rows,
                     zbuf, acc, *sems):
    c = lax.axis_index("c")
    s = lax.axis_index("s")

    pltpu.sync_copy(ei_hbm.at[0, s], sidx)
    pltpu.sync_copy(ei_hbm.at[1, s], didx)

    _zero_vmem_2d(zbuf, ZR, DH)

    @pl.loop(0, RPT // ZR)
    def _(t):
        pltpu.sync_copy(zbuf, acc.at[pl.ds(s * RPT + t * ZR, ZR)])

    plsc.subcore_barrier()

    for i in range(NB):
        pltpu.async_copy(m_hbm.at[c].at[sidx.at[i]], rows.at[i], sems[i])

    @pl.loop(0, NRND)
    def _(t):
        for i in range(NB):
            j = t * NB + i
            pltpu.make_async_copy(m_hbm.at[c].at[sidx.at[0]],
                                  rows.at[i], sems[i]).wait()
            pltpu.async_copy(rows.at[i], acc.at[didx.at[j]], sems[i],
                             add=True)
            pltpu.make_async_copy(m_hbm.at[c].at[sidx.at[0]],
                                  rows.at[i], sems[i]).wait()
            jn = j + NB

            @pl.when(jn < NCH2)
            def _():
                pltpu.async_copy(m_hbm.at[c].at[sidx.at[jn]],
                                 rows.at[i], sems[i])

    plsc.subcore_barrier()
    pltpu.sync_copy(acc.at[pl.ds(s * RPT, RPT)], out_hbm.at[c, pl.ds(s * RPT, RPT)])


def _dot(a, b):
    return jnp.dot(a, b)


def _expproj(t):
    n = jnp.maximum(jnp.sqrt(jnp.sum(t * t, axis=1, keepdims=True)), 1e-15)
    y = jnp.tanh(n) * t / n
    n2 = jnp.maximum(jnp.sqrt(jnp.sum(y * y, axis=1, keepdims=True)), 1e-15)
    maxn = 1.0 - 4e-3
    return jnp.where(n2 > maxn, y * (maxn / n2), y)


R = 1000
G = N // R
RD = 1024
GD = NP // RD


def _tc_dinv_body(hist_ref, dinv_ref):
    ones = jnp.ones((NW, 1), _f32)
    deg = lax.dot_general(hist_ref[...], ones, (((0,), (0,)), ((), ())),
                          precision=lax.Precision.HIGHEST,
                          preferred_element_type=_f32)
    dinv_ref[...] = lax.rsqrt(deg + 1.0)


def _tc_dinv(hist):
    return pl.pallas_call(
        _tc_dinv_body,
        grid=(GD,),
        in_specs=[pl.BlockSpec((NW, RD), lambda i: (0, i))],
        out_specs=pl.BlockSpec((RD, 1), lambda i: (i, 0)),
        out_shape=jax.ShapeDtypeStruct((NP, 1), _f32),
    )(hist)


def _tc_pre_body(x_ref, dinv_ref, w1_ref, m1_ref):
    x = x_ref[...]
    n = jnp.maximum(jnp.sqrt(jnp.sum(x * x, axis=1, keepdims=True)), 1e-15)
    z = jnp.clip(n, -1.0 + 1e-7, 1.0 - 1e-7)
    artanh = 0.5 * jnp.log((1.0 + z) / (1.0 - z))
    h0 = artanh * x / n
    mh = dinv_ref[...] * _dot(h0, w1_ref[...])
    m1_ref[0] = mh[:, :DH]
    m1_ref[1] = mh[:, DH:]


def _tc_pre(x, dinv, W1):
    return pl.pallas_call(
        _tc_pre_body,
        grid=(G,),
        in_specs=[
            pl.BlockSpec((R, D), lambda i: (i, 0)),
            pl.BlockSpec((R, 1), lambda i: (i, 0)),
            pl.BlockSpec((D, D), lambda i: (0, 0)),
        ],
        out_specs=pl.BlockSpec((NC, R, DH), lambda i: (0, i, 0)),
        out_shape=jax.ShapeDtypeStruct((NC, N, DH), _f32),
    )(x, dinv, W1)


def _cat2(r):
    return jnp.concatenate([r[0], r[1]], axis=1)


def _tc_layer_body(acc_ref, m_ref, dinv_ref, w_ref, h_ref, mn_ref):
    dinv = dinv_ref[...]
    h = jnp.maximum(dinv * (_cat2(acc_ref) + _cat2(m_ref)), 0.0)
    h_ref[...] = h
    mh = dinv * _dot(h, w_ref[...])
    mn_ref[0] = mh[:, :DH]
    mn_ref[1] = mh[:, DH:]


def _tc_layer(acc, m, dinv, Wn):
    return pl.pallas_call(
        _tc_layer_body,
        grid=(G,),
        in_specs=[
            pl.BlockSpec((NC, R, DH), lambda i: (0, i, 0)),
            pl.BlockSpec((NC, R, DH), lambda i: (0, i, 0)),
            pl.BlockSpec((R, 1), lambda i: (i, 0)),
            pl.BlockSpec((D, D), lambda i: (0, 0)),
        ],
        out_specs=[
            pl.BlockSpec((R, D), lambda i: (i, 0)),
            pl.BlockSpec((NC, R, DH), lambda i: (0, i, 0)),
        ],
        out_shape=[
            jax.ShapeDtypeStruct((N, D), _f32),
            jax.ShapeDtypeStruct((NC, N, DH), _f32),
        ],
    )(acc, m, dinv, Wn)


def _tc_final_body(acc_ref, m_ref, dinv_ref, h1_ref, h2_ref, b_ref, p1_ref,
                   p2_ref, hypn_ref, gtan_ref):
    i = pl.program_id(0)
    dinv = dinv_ref[...]
    h3 = jnp.maximum(dinv * (_cat2(acc_ref) + _cat2(m_ref)), 0.0)
    ncat = jnp.concatenate([h1_ref[...], h2_ref[...], h3], axis=1)
    t = _dot(jnp.maximum(_dot(ncat, p1_ref[...]), 0.0), p2_ref[...])
    hypn_ref[...] = _expproj(t)

    b = b_ref[0]
    rows = lax.broadcasted_iota(jnp.int32, (B, R), 0)
    oh = (rows == b).astype(_f32)
    gpart = _dot(oh, ncat)

    @pl.when(i == 0)
    def _():
        gtan_ref[...] = gpart

    @pl.when(i > 0)
    def _():
        gtan_ref[...] += gpart


def _tc_final(acc, m3, dinv, h1, h2, batch3, P1, P2):
    return pl.pallas_call(
        _tc_final_body,
        grid=(G,),
        in_specs=[
            pl.BlockSpec((NC, R, DH), lambda i: (0, i, 0)),
            pl.BlockSpec((NC, R, DH), lambda i: (0, i, 0)),
            pl.BlockSpec((R, 1), lambda i: (i, 0)),
            pl.BlockSpec((R, D), lambda i: (i, 0)),
            pl.BlockSpec((R, D), lambda i: (i, 0)),
            pl.BlockSpec((1, 1, R), lambda i: (i, 0, 0)),
            pl.BlockSpec((ED, ED), lambda i: (0, 0)),
            pl.BlockSpec((ED, ED), lambda i: (0, 0)),
        ],
        out_specs=[
            pl.BlockSpec((R, ED), lambda i: (i, 0)),
            pl.BlockSpec((B, ED), lambda i: (0, 0)),
        ],
        out_shape=[
            jax.ShapeDtypeStruct((N, ED), _f32),
            jax.ShapeDtypeStruct((B, ED), _f32),
        ],
    )(acc, m3, dinv, h1, h2, batch3, P1, P2)


def _tc_head_body(g_ref, p1_ref, p2_ref, out_ref):
    t = _dot(jnp.maximum(_dot(g_ref[...], p1_ref[...]), 0.0), p2_ref[...])
    out_ref[...] = _expproj(t)


def _tc_head(gtan, P1, P2):
    return pl.pallas_call(
        _tc_head_body,
        out_shape=jax.ShapeDtypeStruct((B, ED), _f32),
    )(gtan, P1, P2)


def kernel(x, edge_index, batch, W1, W2, W3, P1, P2):
    ei4 = edge_index.reshape(2, NS, NCH2, K)
    ei4d = edge_index.reshape(2, NW, NCHUNK, K)
    batch3 = batch.reshape(G, 1, R)

    hist = _sc_degree(ei4d)
    dinv = _tc_dinv(hist)
    m1 = _tc_pre(x, dinv, W1)
    acc1 = _sc_scatter(ei4, m1)
    h1, m2 = _tc_layer(acc1, m1, dinv, W2)
    acc2 = _sc_scatter(ei4, m2)
    h2, m3 = _tc_layer(acc2, m2, dinv, W3)
    acc3 = _sc_scatter(ei4, m3)
    hyp_n, gtan = _tc_final(acc3, m3, dinv, h1, h2, batch3, P1, P2)
    hyp_g = _tc_head(gtan, P1, P2)
    return (hyp_g, hyp_n)


# speedup vs baseline: 1.0865x; 1.0180x over previous
import functools

import jax
import jax.numpy as jnp
from jax import lax
from jax.experimental import pallas as pl
from jax.experimental.pallas import tpu as pltpu
from jax.experimental.pallas import tpu_sc as plsc

N = 10000
E = 320000
D = 128
B = 64
ED = 3 * D

NC = 2
NS = 16
NW = NC * NS
EW = E // NW
K = 80
NCHUNK = EW // K
DH = D // NC
ET = E // NS
NCH2 = ET // K
NB = 5
NRND = NCH2 // NB
NP = 10240
RPT = NP // NS
ZR = 128

_f32 = jnp.float32


def _sc_mesh():
    return plsc.VectorSubcoreMesh(
        core_axis_name="c", subcore_axis_name="s", num_cores=NC, num_subcores=NS
    )


def _zero_vmem_2d(ref, nrows, ncols):
    z = jnp.zeros((16,), _f32)

    @pl.loop(0, nrows)
    def _(i):
        for j in range(ncols // 16):
            ref[i, pl.ds(j * 16, 16)] = z


@functools.cache
def _sc_degree_kernel():
    return functools.partial(
        pl.kernel,
        out_type=jax.ShapeDtypeStruct((NW, NP), _f32),
        mesh=_sc_mesh(),
        scratch_types=[
            pltpu.VMEM((NCHUNK, K), jnp.int32),
            pltpu.VMEM((NP,), _f32),
        ],
        compiler_params=pltpu.CompilerParams(needs_layout_passes=False),
    )(_sc_degree_body)


def _sc_degree(ei4d):
    return _sc_degree_kernel()(ei4d)


def _sc_degree_body(ei_hbm, out_hbm, didx, hist):
    c = lax.axis_index("c")
    s = lax.axis_index("s")
    wid = c * NS + s

    pltpu.sync_copy(ei_hbm.at[1, wid], didx)

    z = jnp.zeros((16,), _f32)

    @pl.loop(0, NP // 16)
    def _(i):
        hist[pl.ds(i * 16, 16)] = z

    one = jnp.ones((16,), _f32)

    @pl.loop(0, NCHUNK)
    def _(i):
        for j in range(K // 16):
            d16 = didx[i, pl.ds(j * 16, 16)]
            plsc.addupdate_scatter(hist, [d16], one)

    pltpu.sync_copy(hist, out_hbm.at[wid])


@functools.cache
def _sc_scatter_kernel():
    return functools.partial(
        pl.kernel,
        out_type=jax.ShapeDtypeStruct((NC, NP, DH), _f32),
        mesh=_sc_mesh(),
        scratch_types=[
            pltpu.VMEM((NCH2, K), jnp.int32),
            pltpu.VMEM((NCH2, K), jnp.int32),
            pltpu.VMEM((NB, K, DH), _f32),
            pltpu.VMEM((ZR, DH), _f32),
            pltpu.VMEM_SHARED((NP, DH), _f32),
        ] + [pltpu.SemaphoreType.DMA] * NB,
        compiler_params=pltpu.CompilerParams(use_tc_tiling_on_sc=False),
    )(_sc_scatter_body)


def _sc_scatter(ei4, m2):
    return _sc_scatter_kernel()(ei4, m2)


def _sc_scatter_body(ei_hbm, m_hbm, out_hbm, sidx, didx, rows,
                     zbuf, acc, *sems):
    c = lax.axis_index("c")
    s = lax.axis_index("s")

    pltpu.sync_copy(ei_hbm.at[0, s], sidx)
    pltpu.sync_copy(ei_hbm.at[1, s], didx)

    _zero_vmem_2d(zbuf, ZR, DH)

    @pl.loop(0, RPT // ZR)
    def _(t):
        pltpu.sync_copy(zbuf, acc.at[pl.ds(s * RPT + t * ZR, ZR)])

    plsc.subcore_barrier()

    for i in range(NB):
        pltpu.async_copy(m_hbm.at[c].at[sidx.at[i]], rows.at[i], sems[i])

    @pl.loop(0, NRND)
    def _(t):
        for i in range(NB):
            j = t * NB + i
            pltpu.make_async_copy(m_hbm.at[c].at[sidx.at[0]],
                                  rows.at[i], sems[i]).wait()
            pltpu.async_copy(rows.at[i], acc.at[didx.at[j]], sems[i],
                             add=True)
            pltpu.make_async_copy(m_hbm.at[c].at[sidx.at[0]],
                                  rows.at[i], sems[i]).wait()
            jn = j + NB

            @pl.when(jn < NCH2)
            def _():
                pltpu.async_copy(m_hbm.at[c].at[sidx.at[jn]],
                                 rows.at[i], sems[i])

    plsc.subcore_barrier()
    pltpu.sync_copy(acc.at[pl.ds(s * RPT, RPT)], out_hbm.at[c, pl.ds(s * RPT, RPT)])


def _dot(a, b):
    return jnp.dot(a, b)


def _expproj(t):
    n = jnp.maximum(jnp.sqrt(jnp.sum(t * t, axis=1, keepdims=True)), 1e-15)
    y = jnp.tanh(n) * t / n
    n2 = jnp.maximum(jnp.sqrt(jnp.sum(y * y, axis=1, keepdims=True)), 1e-15)
    maxn = 1.0 - 4e-3
    return jnp.where(n2 > maxn, y * (maxn / n2), y)


R = 2000
G = N // R
RD = 1024
GD = NP // RD


def _tc_dinv_body(hist_ref, dinv_ref):
    ones = jnp.ones((NW, 1), _f32)
    deg = lax.dot_general(hist_ref[...], ones, (((0,), (0,)), ((), ())),
                          precision=lax.Precision.HIGHEST,
                          preferred_element_type=_f32)
    dinv_ref[...] = lax.rsqrt(deg + 1.0)


def _tc_dinv(hist):
    return pl.pallas_call(
        _tc_dinv_body,
        grid=(GD,),
        in_specs=[pl.BlockSpec((NW, RD), lambda i: (0, i))],
        out_specs=pl.BlockSpec((RD, 1), lambda i: (i, 0)),
        out_shape=jax.ShapeDtypeStruct((NP, 1), _f32),
    )(hist)


def _tc_pre_body(x_ref, dinv_ref, w1_ref, m1_ref):
    x = x_ref[...]
    n = jnp.maximum(jnp.sqrt(jnp.sum(x * x, axis=1, keepdims=True)), 1e-15)
    z = jnp.clip(n, -1.0 + 1e-7, 1.0 - 1e-7)
    artanh = 0.5 * jnp.log((1.0 + z) / (1.0 - z))
    h0 = artanh * x / n
    mh = dinv_ref[...] * _dot(h0, w1_ref[...])
    m1_ref[0] = mh[:, :DH]
    m1_ref[1] = mh[:, DH:]


def _tc_pre(x, dinv, W1):
    return pl.pallas_call(
        _tc_pre_body,
        grid=(G,),
        in_specs=[
            pl.BlockSpec((R, D), lambda i: (i, 0)),
            pl.BlockSpec((R, 1), lambda i: (i, 0)),
            pl.BlockSpec((D, D), lambda i: (0, 0)),
        ],
        out_specs=pl.BlockSpec((NC, R, DH), lambda i: (0, i, 0)),
        out_shape=jax.ShapeDtypeStruct((NC, N, DH), _f32),
    )(x, dinv, W1)


def _cat2(r):
    return jnp.concatenate([r[0], r[1]], axis=1)


def _tc_layer_body(acc_ref, m_ref, dinv_ref, w_ref, h_ref, mn_ref):
    dinv = dinv_ref[...]
    h = jnp.maximum(dinv * (_cat2(acc_ref) + _cat2(m_ref)), 0.0)
    h_ref[...] = h
    mh = dinv * _dot(h, w_ref[...])
    mn_ref[0] = mh[:, :DH]
    mn_ref[1] = mh[:, DH:]


def _tc_layer(acc, m, dinv, Wn):
    return pl.pallas_call(
        _tc_layer_body,
        grid=(G,),
        in_specs=[
            pl.BlockSpec((NC, R, DH), lambda i: (0, i, 0)),
            pl.BlockSpec((NC, R, DH), lambda i: (0, i, 0)),
            pl.BlockSpec((R, 1), lambda i: (i, 0)),
            pl.BlockSpec((D, D), lambda i: (0, 0)),
        ],
        out_specs=[
            pl.BlockSpec((R, D), lambda i: (i, 0)),
            pl.BlockSpec((NC, R, DH), lambda i: (0, i, 0)),
        ],
        out_shape=[
            jax.ShapeDtypeStruct((N, D), _f32),
            jax.ShapeDtypeStruct((NC, N, DH), _f32),
        ],
    )(acc, m, dinv, Wn)


def _tc_final_body(acc_ref, m_ref, dinv_ref, h1_ref, h2_ref, b_ref, p1_ref,
                   p2_ref, hypn_ref, gtan_ref, hypg_ref):
    i = pl.program_id(0)

    @pl.when(i < G)
    def _():
        dinv = dinv_ref[...]
        h3 = jnp.maximum(dinv * (_cat2(acc_ref) + _cat2(m_ref)), 0.0)
        ncat = jnp.concatenate([h1_ref[...], h2_ref[...], h3], axis=1)
        t = _dot(jnp.maximum(_dot(ncat, p1_ref[...]), 0.0), p2_ref[...])
        hypn_ref[...] = _expproj(t)

        b = b_ref[0]
        rows = lax.broadcasted_iota(jnp.int32, (B, R), 0)
        oh = (rows == b).astype(_f32)
        gpart = _dot(oh, ncat)

        @pl.when(i == 0)
        def _():
            gtan_ref[...] = gpart

        @pl.when(i > 0)
        def _():
            gtan_ref[...] += gpart

    @pl.when(i == G)
    def _():
        g = gtan_ref[...]
        tg = _dot(jnp.maximum(_dot(g, p1_ref[...]), 0.0), p2_ref[...])
        hypg_ref[...] = _expproj(tg)


def _tc_final(acc, m3, dinv, h1, h2, batch3, P1, P2):
    cl = lambda i: jnp.minimum(i, G - 1)
    return pl.pallas_call(
        _tc_final_body,
        grid=(G + 1,),
        in_specs=[
            pl.BlockSpec((NC, R, DH), lambda i: (0, cl(i), 0)),
            pl.BlockSpec((NC, R, DH), lambda i: (0, cl(i), 0)),
            pl.BlockSpec((R, 1), lambda i: (cl(i), 0)),
            pl.BlockSpec((R, D), lambda i: (cl(i), 0)),
            pl.BlockSpec((R, D), lambda i: (cl(i), 0)),
            pl.BlockSpec((1, 1, R), lambda i: (cl(i), 0, 0)),
            pl.BlockSpec((ED, ED), lambda i: (0, 0)),
            pl.BlockSpec((ED, ED), lambda i: (0, 0)),
        ],
        out_specs=[
            pl.BlockSpec((R, ED), lambda i: (cl(i), 0)),
            pl.BlockSpec((B, ED), lambda i: (0, 0)),
            pl.BlockSpec((B, ED), lambda i: (0, 0)),
        ],
        out_shape=[
            jax.ShapeDtypeStruct((N, ED), _f32),
            jax.ShapeDtypeStruct((B, ED), _f32),
            jax.ShapeDtypeStruct((B, ED), _f32),
        ],
    )(acc, m3, dinv, h1, h2, batch3, P1, P2)


def _tc_head_body(g_ref, p1_ref, p2_ref, out_ref):
    t = _dot(jnp.maximum(_dot(g_ref[...], p1_ref[...]), 0.0), p2_ref[...])
    out_ref[...] = _expproj(t)


def _tc_head(gtan, P1, P2):
    return pl.pallas_call(
        _tc_head_body,
        out_shape=jax.ShapeDtypeStruct((B, ED), _f32),
    )(gtan, P1, P2)


def kernel(x, edge_index, batch, W1, W2, W3, P1, P2):
    ei4 = edge_index.reshape(2, NS, NCH2, K)
    ei4d = edge_index.reshape(2, NW, NCHUNK, K)
    batch3 = batch.reshape(G, 1, R)

    hist = _sc_degree(ei4d)
    dinv = _tc_dinv(hist)
    m1 = _tc_pre(x, dinv, W1)
    acc1 = _sc_scatter(ei4, m1)
    h1, m2 = _tc_layer(acc1, m1, dinv, W2)
    acc2 = _sc_scatter(ei4, m2)
    h2, m3 = _tc_layer(acc2, m2, dinv, W3)
    acc3 = _sc_scatter(ei4, m3)
    hyp_n, _gtan, hyp_g = _tc_final(acc3, m3, dinv, h1, h2, batch3, P1, P2)
    return (hyp_g, hyp_n)
